# baseline (speedup 1.0000x reference)
import functools

import jax
import jax.numpy as jnp
from jax import lax
from jax.experimental import pallas as pl
import numpy as np

TOPK = 8
INTERP = False
_ORDER = [h * 7 + w for w in range(7) for h in range(7)]
_DN = (((1,), (1,)), ((), ()))


def _pool_body(x_ref, o_ref, *, rb):
    xt = jnp.transpose(x_ref[...], (1, 0))
    a = xt[_ORDER[0]:_ORDER[0] + 1, :]
    for s in _ORDER[1:]:
        a = a + xt[s:s + 1, :]
    o_ref[...] = (a * np.float32(1.0 / 49.0)).reshape(1, 1, rb)


def _mlp_body(p_ref, noise_ref, W1_ref, b1_ref, W2_ref, b2_ref, W3_ref,
              b3_ref, W4_ref, b4_ref, Wu_ref, bu_ref, Wc_ref, bc_ref,
              rw_ref, idx_ref, cls_ref, usage_ref, l2_ref, *, inv_b):
    f32 = jnp.float32

    def mm(x, w):
        return lax.dot_general(x, w, _DN, preferred_element_type=f32)

    x = p_ref[...]
    h = jax.nn.relu(mm(x, W1_ref[...]) + b1_ref[...])
    h = jax.nn.relu(mm(h.astype(jnp.bfloat16), W2_ref[...]) + b2_ref[...])
    h = jax.nn.relu(mm(h, W3_ref[...]) + b3_ref[...])
    h = mm(h.astype(jnp.bfloat16), W4_ref[...]) + b4_ref[...]

    scores = mm(h, Wu_ref[...]) + bu_ref[...] + noise_ref[...]
    cls_ref[...] = mm(h, Wc_ref[...]) + bc_ref[...]

    tb, e = scores.shape
    cols = lax.broadcasted_iota(jnp.int32, (tb, e), 1)
    work = scores
    selected = jnp.zeros((tb, e), jnp.bool_)
    neg_inf = jnp.float32(-jnp.inf)
    idx_cols = []
    for _ in range(TOPK):
        m = jnp.max(work, axis=1, keepdims=True)
        idx = jnp.min(jnp.where(work == m, cols, e), axis=1, keepdims=True)
        idx_cols.append(idx)
        hit = cols == idx
        selected = jnp.logical_or(selected, hit)
        work = jnp.where(hit, neg_inf, work)
    idx_ref[...] = jnp.concatenate(idx_cols, axis=1)

    rowmax = jnp.max(scores, axis=1, keepdims=True)
    ex = jnp.where(selected, jnp.exp(scores - rowmax), 0.0)
    rw = ex / jnp.sum(ex, axis=1, keepdims=True)
    rw_ref[...] = rw

    part = jnp.sum(rw, axis=0, keepdims=True) * inv_b
    first = pl.program_id(0) == 0

    @pl.when(first)
    def _():
        usage_ref[...] = part
        l2 = jnp.float32(0.0)
        for r in (W1_ref, b1_ref, W2_ref, b2_ref, W3_ref, b3_ref, W4_ref,
                  b4_ref, Wu_ref, bu_ref, Wc_ref, bc_ref):
            v = r[...]
            l2 = l2 + jnp.sqrt(jnp.sum(v * v))
        l2_ref[...] = jnp.full((1, 1), 0.01, f32) * l2

    @pl.when(jnp.logical_not(first))
    def _():
        usage_ref[...] = usage_ref[...] + part


def kernel(inputs, W1, b1, W2, b2, W3, b3, W4, b4, Wu, bu, Wc, bc):
    B, C, H, W = inputs.shape
    S = H * W
    E = Wu.shape[0]
    L = Wc.shape[0]
    f32 = jnp.float32

    noise = jax.random.normal(jax.random.key(1234), (B, E), f32) * 0.01

    x2 = inputs.reshape(B * C, S)
    rb = 4096
    nblk = (B * C) // rb
    pooled = pl.pallas_call(
        functools.partial(_pool_body, rb=rb),
        grid=(nblk,),
        in_specs=[pl.BlockSpec((rb, S), lambda i: (i, 0))],
        out_specs=pl.BlockSpec((1, 1, rb), lambda i: (i, 0, 0)),
        out_shape=jax.ShapeDtypeStruct((nblk, 1, rb), f32),
        interpret=INTERP,
    )(x2).reshape(B, C)

    TB = 256
    grid = (B // TB,)
    row_spec = lambda n: pl.BlockSpec((TB, n), lambda i: (i, 0))
    full = lambda a: pl.BlockSpec(a.shape, lambda i: (0,) * a.ndim)
    b1r, b2r, b3r, b4r = (b.reshape(1, -1) for b in (b1, b2, b3, b4))
    bur, bcr = bu.reshape(1, -1), bc.reshape(1, -1)

    rw, idx, cls, usage, l2 = pl.pallas_call(
        functools.partial(_mlp_body, inv_b=float(1.0 / B)),
        grid=grid,
        in_specs=[row_spec(C), row_spec(E)] + [
            full(a) for a in (W1, b1r, W2, b2r, W3, b3r, W4, b4r,
                              Wu, bur, Wc, bcr)],
        out_specs=(row_spec(E), row_spec(TOPK), row_spec(L),
                   pl.BlockSpec((1, E), lambda i: (0, 0)),
                   pl.BlockSpec((1, 1), lambda i: (0, 0))),
        out_shape=(jax.ShapeDtypeStruct((B, E), f32),
                   jax.ShapeDtypeStruct((B, TOPK), jnp.int32),
                   jax.ShapeDtypeStruct((B, L), f32),
                   jax.ShapeDtypeStruct((1, E), f32),
                   jax.ShapeDtypeStruct((1, 1), f32)),
        interpret=INTERP,
    )(pooled, noise, W1, b1r, W2, b2r, W3, b3r, W4, b4r, Wu, bur, Wc, bcr)

    return (rw, idx, cls, l2.reshape(()), usage.reshape(E))


# speedup vs baseline: 3.1884x; 3.1884x over previous
import functools

import jax
import jax.numpy as jnp
from jax import lax
from jax.experimental import pallas as pl
import numpy as np

TOPK = 8
INTERP = False
_ORDER = [h * 7 + w for w in range(7) for h in range(7)]
_DN = (((1,), (1,)), ((), ()))


def _pool_body(x_ref, o_ref):
    a = x_ref[_ORDER[0]]
    for s in _ORDER[1:]:
        a = a + x_ref[s]
    o_ref[...] = a * np.float32(1.0 / 49.0)


def _mlp_body(p_ref, noise_ref, W1_ref, b1_ref, W2_ref, b2_ref, W3_ref,
              b3_ref, W4_ref, b4_ref, Wu_ref, bu_ref, Wc_ref, bc_ref,
              rw_ref, idx_ref, cls_ref, usage_ref, l2_ref, *, inv_b):
    f32 = jnp.float32

    def mm(x, w):
        return lax.dot_general(x, w, _DN, preferred_element_type=f32)

    x = p_ref[...]
    h = jax.nn.relu(mm(x, W1_ref[...]) + b1_ref[...])
    h = jax.nn.relu(mm(h.astype(jnp.bfloat16), W2_ref[...]) + b2_ref[...])
    h = jax.nn.relu(mm(h, W3_ref[...]) + b3_ref[...])
    h = mm(h.astype(jnp.bfloat16), W4_ref[...]) + b4_ref[...]

    scores = mm(h, Wu_ref[...]) + bu_ref[...] + noise_ref[...]
    cls_ref[...] = mm(h, Wc_ref[...]) + bc_ref[...]

    tb, e = scores.shape
    cols = lax.broadcasted_iota(jnp.int32, (tb, e), 1)
    work = scores
    selected = jnp.zeros((tb, e), jnp.bool_)
    neg_inf = jnp.float32(-jnp.inf)
    idx_cols = []
    for _ in range(TOPK):
        m = jnp.max(work, axis=1, keepdims=True)
        idx = jnp.min(jnp.where(work == m, cols, e), axis=1, keepdims=True)
        idx_cols.append(idx)
        hit = cols == idx
        selected = jnp.logical_or(selected, hit)
        work = jnp.where(hit, neg_inf, work)
    idx_ref[...] = jnp.concatenate(idx_cols, axis=1)

    rowmax = jnp.max(scores, axis=1, keepdims=True)
    ex = jnp.where(selected, jnp.exp(scores - rowmax), 0.0)
    rw = ex / jnp.sum(ex, axis=1, keepdims=True)
    rw_ref[...] = rw

    part = jnp.sum(rw, axis=0, keepdims=True) * inv_b
    first = pl.program_id(0) == 0

    @pl.when(first)
    def _():
        usage_ref[...] = part
        l2 = jnp.float32(0.0)
        for r in (W1_ref, b1_ref, W2_ref, b2_ref, W3_ref, b3_ref, W4_ref,
                  b4_ref, Wu_ref, bu_ref, Wc_ref, bc_ref):
            v = r[...]
            l2 = l2 + jnp.sqrt(jnp.sum(v * v))
        l2_ref[...] = jnp.full((1, 1), 0.01, f32) * l2

    @pl.when(jnp.logical_not(first))
    def _():
        usage_ref[...] = usage_ref[...] + part


def kernel(inputs, W1, b1, W2, b2, W3, b3, W4, b4, Wu, bu, Wc, bc):
    B, C, H, W = inputs.shape
    S = H * W
    E = Wu.shape[0]
    L = Wc.shape[0]
    f32 = jnp.float32

    noise = jax.random.normal(jax.random.key(1234), (B, E), f32) * 0.01

    lanes = 128
    sb_total = (B * C) // lanes
    x_pl = jnp.transpose(inputs, (2, 3, 0, 1)).reshape(S, sb_total, lanes)
    SB = 512
    nblk = sb_total // SB
    pooled = pl.pallas_call(
        _pool_body,
        grid=(nblk,),
        in_specs=[pl.BlockSpec((S, SB, lanes), lambda i: (0, i, 0))],
        out_specs=pl.BlockSpec((SB, lanes), lambda i: (i, 0)),
        out_shape=jax.ShapeDtypeStruct((sb_total, lanes), f32),
        interpret=INTERP,
    )(x_pl).reshape(B, C)

    TB = 256
    grid = (B // TB,)
    row_spec = lambda n: pl.BlockSpec((TB, n), lambda i: (i, 0))
    full = lambda a: pl.BlockSpec(a.shape, lambda i: (0,) * a.ndim)
    b1r, b2r, b3r, b4r = (b.reshape(1, -1) for b in (b1, b2, b3, b4))
    bur, bcr = bu.reshape(1, -1), bc.reshape(1, -1)

    rw, idx, cls, usage, l2 = pl.pallas_call(
        functools.partial(_mlp_body, inv_b=float(1.0 / B)),
        grid=grid,
        in_specs=[row_spec(C), row_spec(E)] + [
            full(a) for a in (W1, b1r, W2, b2r, W3, b3r, W4, b4r,
                              Wu, bur, Wc, bcr)],
        out_specs=(row_spec(E), row_spec(TOPK), row_spec(L),
                   pl.BlockSpec((1, E), lambda i: (0, 0)),
                   pl.BlockSpec((1, 1), lambda i: (0, 0))),
        out_shape=(jax.ShapeDtypeStruct((B, E), f32),
                   jax.ShapeDtypeStruct((B, TOPK), jnp.int32),
                   jax.ShapeDtypeStruct((B, L), f32),
                   jax.ShapeDtypeStruct((1, E), f32),
                   jax.ShapeDtypeStruct((1, 1), f32)),
        interpret=INTERP,
    )(pooled, noise, W1, b1r, W2, b2r, W3, b3r, W4, b4r, Wu, bur, Wc, bcr)

    return (rw, idx, cls, l2.reshape(()), usage.reshape(E))


# speedup vs baseline: 8.3564x; 2.6209x over previous
import functools

import jax
import jax.numpy as jnp
from jax import lax
from jax.experimental import pallas as pl
from jax.experimental.pallas import tpu as pltpu
import numpy as np

TOPK = 8
INTERP = False
_NP = 49
_DN = (((1,), (1,)), ((), ()))


def _body(x_ref, noise_ref, W1_ref, b1_ref, W2_ref, b2_ref, W3_ref,
          b3_ref, W4_ref, b4_ref, Wu_ref, bu_ref, Wc_ref, bc_ref,
          rw_ref, idx_ref, cls_ref, usage_ref, l2_ref, acc_ref, *, inv_b):
    f32 = jnp.float32
    i = pl.program_id(0)

    @pl.when(i == 0)
    def _():
        acc_ref[...] = x_ref[0]
        l2 = jnp.float32(0.0)
        for r in (W1_ref, b1_ref, W2_ref, b2_ref, W3_ref, b3_ref, W4_ref,
                  b4_ref, Wu_ref, bu_ref, Wc_ref, bc_ref):
            v = r[...]
            l2 = l2 + jnp.sqrt(jnp.sum(v * v))
        l2_ref[...] = jnp.full((1, 1), 0.01, f32) * l2

    @pl.when(jnp.logical_and(i > 0, i < _NP))
    def _():
        acc_ref[...] = acc_ref[...] + x_ref[0]

    @pl.when(i == _NP)
    def _():
        def mm(x, w):
            return lax.dot_general(x, w, _DN, preferred_element_type=f32)

        pooled = acc_ref[...] * np.float32(1.0 / 49.0)
        h = jax.nn.relu(mm(pooled, W1_ref[...]) + b1_ref[...])
        h = jax.nn.relu(mm(h.astype(jnp.bfloat16), W2_ref[...]) + b2_ref[...])
        h = jax.nn.relu(mm(h, W3_ref[...]) + b3_ref[...])
        h = mm(h.astype(jnp.bfloat16), W4_ref[...]) + b4_ref[...]

        scores = mm(h, Wu_ref[...]) + bu_ref[...] + noise_ref[...]
        cls_ref[...] = mm(h, Wc_ref[...]) + bc_ref[...]

        tb, e = scores.shape
        cols = lax.broadcasted_iota(jnp.int32, (tb, e), 1)
        work = scores
        selected = jnp.zeros((tb, e), jnp.bool_)
        neg_inf = jnp.float32(-jnp.inf)
        idx_cols = []
        for _ in range(TOPK):
            m = jnp.max(work, axis=1, keepdims=True)
            idx = jnp.min(jnp.where(work == m, cols, e), axis=1, keepdims=True)
            idx_cols.append(idx)
            hit = cols == idx
            selected = jnp.logical_or(selected, hit)
            work = jnp.where(hit, neg_inf, work)
        idx_ref[...] = jnp.concatenate(idx_cols, axis=1)

        rowmax = jnp.max(scores, axis=1, keepdims=True)
        ex = jnp.where(selected, jnp.exp(scores - rowmax), 0.0)
        rw = ex / jnp.sum(ex, axis=1, keepdims=True)
        rw_ref[...] = rw
        usage_ref[...] = jnp.sum(rw, axis=0, keepdims=True) * inv_b


def kernel(inputs, W1, b1, W2, b2, W3, b3, W4, b4, Wu, bu, Wc, bc):
    B, C, H, W = inputs.shape
    S = H * W
    E = Wu.shape[0]
    L = Wc.shape[0]
    f32 = jnp.float32

    noise = jax.random.normal(jax.random.key(1234), (B, E), f32) * 0.01
    x_pl = jnp.transpose(inputs, (2, 3, 0, 1)).reshape(S, B, C)

    def x_map(i):
        j = jnp.minimum(i, _NP - 1)
        return ((j % 7) * 7 + j // 7, 0, 0)

    full = lambda a: pl.BlockSpec(a.shape, lambda i: (0,) * a.ndim)
    b1r, b2r, b3r, b4r = (b.reshape(1, -1) for b in (b1, b2, b3, b4))
    bur, bcr = bu.reshape(1, -1), bc.reshape(1, -1)

    rw, idx, cls, usage, l2 = pl.pallas_call(
        functools.partial(_body, inv_b=float(1.0 / B)),
        grid=(_NP + 1,),
        in_specs=[pl.BlockSpec((1, B, C), x_map), full(noise)] + [
            full(a) for a in (W1, b1r, W2, b2r, W3, b3r, W4, b4r,
                              Wu, bur, Wc, bcr)],
        out_specs=(full(jnp.zeros((B, E))), full(jnp.zeros((B, TOPK))),
                   full(jnp.zeros((B, L))),
                   pl.BlockSpec((1, E), lambda i: (0, 0)),
                   pl.BlockSpec((1, 1), lambda i: (0, 0))),
        out_shape=(jax.ShapeDtypeStruct((B, E), f32),
                   jax.ShapeDtypeStruct((B, TOPK), jnp.int32),
                   jax.ShapeDtypeStruct((B, L), f32),
                   jax.ShapeDtypeStruct((1, E), f32),
                   jax.ShapeDtypeStruct((1, 1), f32)),
        scratch_shapes=[pltpu.VMEM((B, C), f32)],
        interpret=INTERP,
    )(x_pl, noise, W1, b1r, W2, b2r, W3, b3r, W4, b4r, Wu, bur, Wc, bcr)

    return (rw, idx, cls, l2.reshape(()), usage.reshape(E))
